# baseline (speedup 1.0000x reference)
import functools

import jax
import jax.numpy as jnp
import numpy as np
from jax import lax
from jax.experimental import pallas as pl
from jax.experimental.pallas import tpu as pltpu
from jax.experimental.pallas import tpu_sc as plsc

_T = 256
_RT = 1024
_NW = 32
_BATCH = 64


def _gumbel(n, e):
    u = jax.random.uniform(jax.random.key(42), (n, e), minval=1e-20,
                           maxval=1.0, dtype=jnp.float32)
    return -jnp.log(-jnp.log(u))


@functools.lru_cache(maxsize=None)
def _gumbel_const(n, e):
    try:
        with jax.ensure_compile_time_eval():
            return np.asarray(_gumbel(n, e))
    except Exception:
        return None


def _router_body(x_ref, wg_ref, bg_ref, g_ref, tril_ref, eid_ref, gate_ref,
                 rank_ref, ts_ref, te_ref, x16_ref, cnt):
    i = pl.program_id(0)
    E = wg_ref.shape[1]
    z = jnp.dot(x_ref[...], wg_ref[...], preferred_element_type=jnp.float32)
    z = z + bg_ref[...] + g_ref[...]
    m = jnp.max(z, axis=1, keepdims=True)
    e = jnp.exp(z - m)
    ssum = jnp.sum(e, axis=1, keepdims=True)
    y = e / ssum
    my = jnp.max(y, axis=1, keepdims=True)
    iota = lax.broadcasted_iota(jnp.int32, z.shape, 1)
    idx = jnp.min(jnp.where(y == my, iota, E), axis=1)
    s = jnp.max(y, axis=1)
    gate = (1.0 + s) - s

    oh16 = (idx[:, None] == iota).astype(jnp.bfloat16)
    onehot = oh16.astype(jnp.float32)
    rank_incl = jnp.dot(tril_ref[...], oh16,
                        preferred_element_type=jnp.float32)
    pos_in_blk = jnp.sum(rank_incl * onehot, axis=1) - 1.0

    @pl.when(i == 0)
    def _():
        cnt[...] = jnp.zeros_like(cnt)

    base = cnt[...]
    grank = pos_in_blk + jnp.sum(onehot * base, axis=1)
    newc = base + jnp.sum(onehot, axis=0, keepdims=True)
    cnt[...] = newc

    @pl.when(i == pl.num_programs(0) - 1)
    def _():
        G = te_ref.shape[1]
        tiles_e = (newc.astype(jnp.int32) + (_T - 1)) // _T
        ia = lax.broadcasted_iota(jnp.int32, (E, E), 0)
        ib = lax.broadcasted_iota(jnp.int32, (E, E), 1)
        strict = (ia < ib).astype(jnp.float32)
        ts = jnp.dot(tiles_e.astype(jnp.float32), strict,
                     preferred_element_type=jnp.float32).astype(jnp.int32)
        cum = ts + tiles_e
        pad = jnp.zeros((1, ts_ref.shape[1] - E), jnp.int32)
        ts_ref[...] = jnp.concatenate([ts, pad], axis=1)
        ta = lax.broadcasted_iota(jnp.int32, (G, E), 0)
        tb = jnp.broadcast_to(cum, (G, E))
        te = jnp.minimum(jnp.sum((ta >= tb).astype(jnp.int32), axis=1),
                         E - 1)
        te_ref[...] = te[None, :]

    eid_ref[...] = idx[:, None]
    gate_ref[...] = gate[:, None]
    rank_ref[...] = grank.astype(jnp.int32)[:, None]
    D2 = x_ref.shape[1] // 2
    xb = x_ref[...].astype(jnp.bfloat16)
    lo = lax.bitcast_convert_type(xb[:, :D2], jnp.uint16).astype(jnp.uint32)
    hi = lax.bitcast_convert_type(xb[:, D2:], jnp.uint16).astype(jnp.uint32)
    x16_ref[...] = ((hi << 16) | lo).astype(jnp.int32)


def _expert_body(te_ref, xs_ref, w1_ref, b1_ref, w2_ref, b2_ref, out_ref):
    D2 = xs_ref.shape[1]
    w = xs_ref[...]
    xlo = lax.bitcast_convert_type(
        (w & 0xFFFF).astype(jnp.uint16), jnp.bfloat16)
    xhi = lax.bitcast_convert_type(
        (w >> 16).astype(jnp.uint16), jnp.bfloat16)
    h = jnp.dot(xlo, w1_ref[0, :D2], preferred_element_type=jnp.float32)
    h = h + jnp.dot(xhi, w1_ref[0, D2:], preferred_element_type=jnp.float32)
    h = jnp.maximum(h + b1_ref[0], 0.0)
    o = jnp.dot(h, w2_ref[0], preferred_element_type=jnp.float32)
    out_ref[...] = o + b2_ref[0, 0, 0]


def _make_sc_scatter(N, D, GT):
    rows_per_w = N // _NW
    K = rows_per_w // _BATCH
    mesh = plsc.VectorSubcoreMesh(core_axis_name="c", subcore_axis_name="s")

    @functools.partial(
        pl.kernel,
        out_type=[
            jax.ShapeDtypeStruct((GT, D), jnp.int32),
            jax.ShapeDtypeStruct((_NW, K, _BATCH), jnp.int32),
        ],
        mesh=mesh,
        scratch_types=[
            pltpu.VMEM((K, _BATCH), jnp.int32),
            pltpu.VMEM((rows_per_w,), jnp.int32),
            pltpu.VMEM((rows_per_w,), jnp.int32),
            pltpu.VMEM((16,), jnp.int32),
            pltpu.VMEM((_BATCH, D), jnp.int32),
            pltpu.VMEM((_BATCH, D), jnp.int32),
            pltpu.SemaphoreType.DMA,
            pltpu.SemaphoreType.DMA,
        ],
    )
    def sc_scatter(x_hbm, eid_hbm, rank_hbm, ts_hbm, out_hbm, pos_hbm,
                   idx_v, eid_v, rank_v, ts_v, buf0, buf1, sem_ld,
                   sem_st):
        w = lax.axis_index("s") * 2 + lax.axis_index("c")
        base = w * rows_per_w
        pltpu.sync_copy(eid_hbm.at[pl.ds(base, rows_per_w)], eid_v)
        pltpu.sync_copy(rank_hbm.at[pl.ds(base, rows_per_w)], rank_v)
        pltpu.sync_copy(ts_hbm, ts_v)
        loads = [None, None]
        scats = [None, None]
        bufs = (buf0, buf1)
        loads[0] = pltpu.async_copy(
            x_hbm.at[pl.ds(base, _BATCH)], bufs[0], sem_ld)
        ts_vec = ts_v[...]
        for c in range(rows_per_w // 16):
            e = eid_v[pl.ds(c * 16, 16)]
            r = rank_v[pl.ds(c * 16, 16)]
            tstart = jnp.zeros((16,), jnp.int32)
            for k in range(8):
                tstart = jnp.where(e == k, ts_vec[k], tstart)
            idx_v[c * 16 // _BATCH, pl.ds((c * 16) % _BATCH, 16)] = (
                tstart * _T + r)
        pltpu.sync_copy(idx_v, pos_hbm.at[w])
        for j in range(K):
            loads[j % 2].wait()
            if j + 1 < K:
                if scats[(j + 1) % 2] is not None:
                    scats[(j + 1) % 2].wait()
                loads[(j + 1) % 2] = pltpu.async_copy(
                    x_hbm.at[pl.ds(base + (j + 1) * _BATCH, _BATCH)],
                    bufs[(j + 1) % 2], sem_ld)
            scats[j % 2] = pltpu.async_copy(
                bufs[j % 2], out_hbm.at[idx_v.at[j]], sem_st)
        scats[(K - 2) % 2].wait()
        scats[(K - 1) % 2].wait()

    return sc_scatter


def kernel(x, Wg, bg, W1, b1, W2, b2):
    B, S, D = x.shape
    N = B * S
    E = W1.shape[0]
    x_flat = x.reshape(N, D)

    gc = _gumbel_const(N, E)
    g = jnp.asarray(gc) if gc is not None else _gumbel(N, E)
    tril = jnp.asarray(np.tri(_RT, dtype=np.float32), dtype=jnp.bfloat16)

    G = N // _T + E
    eid2, gate2, rank2, ts16, teG, x16 = pl.pallas_call(
        _router_body,
        grid=(N // _RT,),
        in_specs=[
            pl.BlockSpec((_RT, D), lambda i: (i, 0)),
            pl.BlockSpec((D, E), lambda i: (0, 0)),
            pl.BlockSpec((1, E), lambda i: (0, 0)),
            pl.BlockSpec((_RT, E), lambda i: (i, 0)),
            pl.BlockSpec((_RT, _RT), lambda i: (0, 0)),
        ],
        out_specs=[
            pl.BlockSpec((_RT, 1), lambda i: (i, 0)),
            pl.BlockSpec((_RT, 1), lambda i: (i, 0)),
            pl.BlockSpec((_RT, 1), lambda i: (i, 0)),
            pl.BlockSpec((1, 16), lambda i: (0, 0)),
            pl.BlockSpec((1, G), lambda i: (0, 0)),
            pl.BlockSpec((_RT, D // 2), lambda i: (i, 0)),
        ],
        out_shape=[
            jax.ShapeDtypeStruct((N, 1), jnp.int32),
            jax.ShapeDtypeStruct((N, 1), jnp.float32),
            jax.ShapeDtypeStruct((N, 1), jnp.int32),
            jax.ShapeDtypeStruct((1, 16), jnp.int32),
            jax.ShapeDtypeStruct((1, G), jnp.int32),
            jax.ShapeDtypeStruct((N, D // 2), jnp.int32),
        ],
        scratch_shapes=[pltpu.VMEM((1, E), jnp.float32)],
    )(x_flat, Wg, bg.reshape(1, E), g, tril)
    gate = gate2[:, 0]
    tile_expert = teG[0]

    x_sorted, pos3 = _make_sc_scatter(N, D // 2, G * _T)(
        x16, eid2.reshape(N), rank2.reshape(N), ts16.reshape(16))
    padded_pos = pos3.reshape(N)

    grid_spec = pltpu.PrefetchScalarGridSpec(
        num_scalar_prefetch=1,
        grid=(G,),
        in_specs=[
            pl.BlockSpec((_T, D // 2), lambda i, te: (i, 0)),
            pl.BlockSpec((1, D, D), lambda i, te: (te[i], 0, 0)),
            pl.BlockSpec((1, 1, D), lambda i, te: (te[i], 0, 0)),
            pl.BlockSpec((1, D, 1), lambda i, te: (te[i], 0, 0)),
            pl.BlockSpec((1, 1, 1), lambda i, te: (te[i], 0, 0)),
        ],
        out_specs=pl.BlockSpec((_T, 1), lambda i, te: (i, 0)),
    )
    out_sorted = pl.pallas_call(
        _expert_body,
        grid_spec=grid_spec,
        out_shape=jax.ShapeDtypeStruct((G * _T, 1), jnp.float32),
    )(tile_expert, x_sorted, W1.astype(jnp.bfloat16), b1.reshape(E, 1, D),
      W2, b2.reshape(E, 1, 1))

    out = out_sorted[padded_pos, 0] * gate
    return out.reshape(B, S, 1)


# speedup vs baseline: 1.0852x; 1.0852x over previous
import functools

import jax
import jax.numpy as jnp
import numpy as np
from jax import lax
from jax.experimental import pallas as pl
from jax.experimental.pallas import tpu as pltpu
from jax.experimental.pallas import tpu_sc as plsc

_T = 256
_RT = 1024
_NW = 32
_BATCH = 64


def _gumbel(n, e):
    u = jax.random.uniform(jax.random.key(42), (n, e), minval=1e-20,
                           maxval=1.0, dtype=jnp.float32)
    return -jnp.log(-jnp.log(u))


@functools.lru_cache(maxsize=None)
def _gumbel_const(n, e):
    try:
        with jax.ensure_compile_time_eval():
            return np.asarray(_gumbel(n, e))
    except Exception:
        return None


def _router_body(x_ref, wg_ref, bg_ref, g_ref, tril_ref, eid_ref, gate_ref,
                 rank_ref, ts_ref, te_ref, x16_ref, cnt):
    i = pl.program_id(0)
    E = wg_ref.shape[1]
    z = jnp.dot(x_ref[...], wg_ref[...], preferred_element_type=jnp.float32)
    z = z + bg_ref[...] + g_ref[...]
    m = jnp.max(z, axis=1, keepdims=True)
    e = jnp.exp(z - m)
    ssum = jnp.sum(e, axis=1, keepdims=True)
    y = e / ssum
    my = jnp.max(y, axis=1, keepdims=True)
    iota = lax.broadcasted_iota(jnp.int32, z.shape, 1)
    idx = jnp.min(jnp.where(y == my, iota, E), axis=1)
    s = jnp.max(y, axis=1)
    gate = (1.0 + s) - s

    oh16 = (idx[:, None] == iota).astype(jnp.bfloat16)
    onehot = oh16.astype(jnp.float32)
    rank_incl = jnp.dot(tril_ref[...], oh16,
                        preferred_element_type=jnp.float32)
    pos_in_blk = jnp.sum(rank_incl * onehot, axis=1) - 1.0

    @pl.when(i == 0)
    def _():
        cnt[...] = jnp.zeros_like(cnt)

    base = cnt[...]
    grank = pos_in_blk + jnp.sum(onehot * base, axis=1)
    newc = base + jnp.sum(onehot, axis=0, keepdims=True)
    cnt[...] = newc

    @pl.when(i == pl.num_programs(0) - 1)
    def _():
        G = te_ref.shape[1]
        tiles_e = (newc.astype(jnp.int32) + (_T - 1)) // _T
        ia = lax.broadcasted_iota(jnp.int32, (E, E), 0)
        ib = lax.broadcasted_iota(jnp.int32, (E, E), 1)
        strict = (ia < ib).astype(jnp.float32)
        ts = jnp.dot(tiles_e.astype(jnp.float32), strict,
                     preferred_element_type=jnp.float32).astype(jnp.int32)
        cum = ts + tiles_e
        pad = jnp.zeros((1, ts_ref.shape[1] - E), jnp.int32)
        ts_ref[...] = jnp.concatenate([ts, pad], axis=1)
        ta = lax.broadcasted_iota(jnp.int32, (G, E), 0)
        tb = jnp.broadcast_to(cum, (G, E))
        te = jnp.minimum(jnp.sum((ta >= tb).astype(jnp.int32), axis=1),
                         E - 1)
        te_ref[...] = te[None, :]

    eid_ref[...] = idx[:, None]
    gate_ref[...] = gate[:, None]
    rank_ref[...] = grank.astype(jnp.int32)[:, None]
    D2 = x_ref.shape[1] // 2
    xb = x_ref[...].astype(jnp.bfloat16)
    lo = lax.bitcast_convert_type(xb[:, :D2], jnp.uint16).astype(jnp.uint32)
    hi = lax.bitcast_convert_type(xb[:, D2:], jnp.uint16).astype(jnp.uint32)
    x16_ref[...] = ((hi << 16) | lo).astype(jnp.int32)


def _expert_body(te_ref, xs_ref, w1_ref, b1_ref, w2_ref, b2_ref, out_ref):
    D2 = xs_ref.shape[1]
    w = xs_ref[...]
    xlo = lax.bitcast_convert_type(
        (w & 0xFFFF).astype(jnp.uint16), jnp.bfloat16)
    xhi = lax.bitcast_convert_type(
        (w >> 16).astype(jnp.uint16), jnp.bfloat16)
    h = jnp.dot(xlo, w1_ref[0, :D2], preferred_element_type=jnp.float32)
    h = h + jnp.dot(xhi, w1_ref[0, D2:], preferred_element_type=jnp.float32)
    h = jnp.maximum(h + b1_ref[0], 0.0)
    o = jnp.dot(h, w2_ref[0], preferred_element_type=jnp.float32)
    out_ref[...] = o + b2_ref[0, 0, 0]


def _make_sc_scatter(N, D, GT):
    rows_per_w = N // _NW
    K = rows_per_w // _BATCH
    mesh = plsc.VectorSubcoreMesh(core_axis_name="c", subcore_axis_name="s")

    @functools.partial(
        pl.kernel,
        out_type=[
            jax.ShapeDtypeStruct((GT, D), jnp.int32),
            jax.ShapeDtypeStruct((_NW, K, _BATCH), jnp.int32),
        ],
        mesh=mesh,
        scratch_types=[
            pltpu.VMEM((K, _BATCH), jnp.int32),
            pltpu.VMEM((rows_per_w,), jnp.int32),
            pltpu.VMEM((rows_per_w,), jnp.int32),
            pltpu.VMEM((16,), jnp.int32),
            pltpu.VMEM((_BATCH, D), jnp.int32),
            pltpu.VMEM((_BATCH, D), jnp.int32),
            pltpu.SemaphoreType.DMA,
            pltpu.SemaphoreType.DMA,
        ],
    )
    def sc_scatter(x_hbm, eid_hbm, rank_hbm, ts_hbm, out_hbm, pos_hbm,
                   idx_v, eid_v, rank_v, ts_v, buf0, buf1, sem_ld,
                   sem_st):
        w = lax.axis_index("s") * 2 + lax.axis_index("c")
        base = w * rows_per_w
        pltpu.sync_copy(eid_hbm.at[pl.ds(base, rows_per_w)], eid_v)
        pltpu.sync_copy(rank_hbm.at[pl.ds(base, rows_per_w)], rank_v)
        pltpu.sync_copy(ts_hbm, ts_v)
        loads = [None, None]
        scats = [None, None]
        bufs = (buf0, buf1)
        loads[0] = pltpu.async_copy(
            x_hbm.at[pl.ds(base, _BATCH)], bufs[0], sem_ld)
        ts_vec = ts_v[...]
        for c in range(rows_per_w // 16):
            e = eid_v[pl.ds(c * 16, 16)]
            r = rank_v[pl.ds(c * 16, 16)]
            tstart = jnp.zeros((16,), jnp.int32)
            for k in range(8):
                tstart = jnp.where(e == k, ts_vec[k], tstart)
            idx_v[c * 16 // _BATCH, pl.ds((c * 16) % _BATCH, 16)] = (
                tstart * _T + r)
        pltpu.sync_copy(idx_v, pos_hbm.at[w])
        for j in range(K):
            loads[j % 2].wait()
            if j + 1 < K:
                if scats[(j + 1) % 2] is not None:
                    scats[(j + 1) % 2].wait()
                loads[(j + 1) % 2] = pltpu.async_copy(
                    x_hbm.at[pl.ds(base + (j + 1) * _BATCH, _BATCH)],
                    bufs[(j + 1) % 2], sem_ld)
            scats[j % 2] = pltpu.async_copy(
                bufs[j % 2], out_hbm.at[idx_v.at[j]], sem_st)
        scats[(K - 2) % 2].wait()
        scats[(K - 1) % 2].wait()

    return sc_scatter


def kernel(x, Wg, bg, W1, b1, W2, b2):
    B, S, D = x.shape
    N = B * S
    E = W1.shape[0]
    x_flat = x.reshape(N, D)

    gc = _gumbel_const(N, E)
    g = jnp.asarray(gc) if gc is not None else _gumbel(N, E)
    tril = jnp.asarray(np.tri(_RT, dtype=np.float32), dtype=jnp.bfloat16)

    G = N // _T + E
    eid2, gate2, rank2, ts16, teG, x16 = pl.pallas_call(
        _router_body,
        grid=(N // _RT,),
        in_specs=[
            pl.BlockSpec((_RT, D), lambda i: (i, 0)),
            pl.BlockSpec((D, E), lambda i: (0, 0)),
            pl.BlockSpec((1, E), lambda i: (0, 0)),
            pl.BlockSpec((_RT, E), lambda i: (i, 0)),
            pl.BlockSpec((_RT, _RT), lambda i: (0, 0)),
        ],
        out_specs=[
            pl.BlockSpec((_RT, 1), lambda i: (i, 0)),
            pl.BlockSpec((_RT, 1), lambda i: (i, 0)),
            pl.BlockSpec((_RT, 1), lambda i: (i, 0)),
            pl.BlockSpec((1, 16), lambda i: (0, 0)),
            pl.BlockSpec((1, G), lambda i: (0, 0)),
            pl.BlockSpec((_RT, D // 2), lambda i: (i, 0)),
        ],
        out_shape=[
            jax.ShapeDtypeStruct((N, 1), jnp.int32),
            jax.ShapeDtypeStruct((N, 1), jnp.float32),
            jax.ShapeDtypeStruct((N, 1), jnp.int32),
            jax.ShapeDtypeStruct((1, 16), jnp.int32),
            jax.ShapeDtypeStruct((1, G), jnp.int32),
            jax.ShapeDtypeStruct((N, D // 2), jnp.int32),
        ],
        scratch_shapes=[pltpu.VMEM((1, E), jnp.float32)],
    )(x_flat, Wg, bg.reshape(1, E), g, tril)
    gate = gate2[:, 0]
    tile_expert = teG[0]

    x_sorted, pos3 = _make_sc_scatter(N, D // 2, G * _T)(
        x16, eid2.reshape(N), rank2.reshape(N), ts16.reshape(16))
    padded_pos = pos3.reshape(N)

    grid_spec = pltpu.PrefetchScalarGridSpec(
        num_scalar_prefetch=1,
        grid=(G,),
        in_specs=[
            pl.BlockSpec((_T, D // 2), lambda i, te: (i, 0)),
            pl.BlockSpec((1, D, D), lambda i, te: (te[i], 0, 0)),
            pl.BlockSpec((1, 1, D), lambda i, te: (te[i], 0, 0)),
            pl.BlockSpec((1, D, 1), lambda i, te: (te[i], 0, 0)),
            pl.BlockSpec((1, 1, 1), lambda i, te: (te[i], 0, 0)),
        ],
        out_specs=pl.BlockSpec((_T, 1), lambda i, te: (i, 0)),
    )
    out_sorted = pl.pallas_call(
        _expert_body,
        grid_spec=grid_spec,
        out_shape=jax.ShapeDtypeStruct((G * _T, 1), jnp.float32),
    )(tile_expert, x_sorted, W1, b1.reshape(E, 1, D), W2,
      b2.reshape(E, 1, 1))

    out = out_sorted[padded_pos, 0] * gate
    return out.reshape(B, S, 1)
